# baseline (speedup 1.0000x reference)
import functools

import jax
import jax.numpy as jnp
from jax import lax
from jax.experimental import pallas as pl
from jax.experimental.pallas import tpu as pltpu
from jax.experimental.pallas import tpu_sc as plsc

E = 8
TOP_K = 2
D_MODEL = 1024
D_FF = 2048
T = 2048
BLK = 256
NB = 24
P = NB * BLK
NEG_INF = -1e30

NC = 2
NS = 16
NW = NC * NS
TPW = T // NW
CH = 32

_HI = jax.lax.Precision.HIGHEST


def _route_kernel(x_ref, rw_ref, d0_ref, d1_ref, g0_ref, g1_ref, eid_ref):
    x = x_ref[...]
    logits = lax.dot_general(x, rw_ref[...], (((1,), (0,)), ((), ())),
                             preferred_element_type=jnp.float32)
    eids = lax.broadcasted_iota(jnp.int32, (T, E), 1)
    m1 = jnp.max(logits, axis=-1, keepdims=True)
    e1 = jnp.min(jnp.where(logits >= m1, eids, E), axis=-1, keepdims=True)
    l2 = jnp.where(eids == e1, NEG_INF, logits)
    m2 = jnp.max(l2, axis=-1, keepdims=True)
    e2 = jnp.min(jnp.where(l2 >= m2, eids, E), axis=-1, keepdims=True)
    r = jnp.exp(m2 - m1)
    g0_ref[...] = 1.0 / (1.0 + r)
    g1_ref[...] = r / (1.0 + r)

    oh0 = (eids == e1).astype(jnp.float32)
    oh1 = (eids == e2).astype(jnp.float32)
    oh = jnp.concatenate([oh0, oh1], axis=1)
    tri = (lax.broadcasted_iota(jnp.int32, (256, 256), 0)
           >= lax.broadcasted_iota(jnp.int32, (256, 256), 1)).astype(jnp.float32)
    parts = []
    carry = jnp.zeros((1, 2 * E), jnp.float32)
    for k in range(T // 256):
        p = lax.dot_general(tri, oh[k * 256:(k + 1) * 256], (((1,), (0,)), ((), ())),
                            precision=_HI, preferred_element_type=jnp.float32) + carry
        parts.append(p)
        carry = p[-1:, :]
    c = jnp.concatenate(parts, axis=0)
    c0 = c[:, :E]
    c1 = c[:, E:]
    c0ex = c0 - oh0
    c1ex = c1 - oh1

    cnt = carry[:, :E] + carry[:, E:]
    pad_cnt = (((cnt.astype(jnp.int32) + (BLK - 1)) >> 8) << 8).astype(jnp.float32)
    m8 = (lax.broadcasted_iota(jnp.int32, (E, E), 0)
          < lax.broadcasted_iota(jnp.int32, (E, E), 1)).astype(jnp.float32)
    off = lax.dot_general(pad_cnt, m8, (((1,), (0,)), ((), ())),
                          precision=_HI, preferred_element_type=jnp.float32)

    rank0 = c0ex + c1ex
    rank1 = c0 + c1ex
    d0_ref[...] = jnp.sum(oh0 * (off + rank0), axis=-1, keepdims=True).astype(jnp.int32)
    d1_ref[...] = jnp.sum(oh1 * (off + rank1), axis=-1, keepdims=True).astype(jnp.int32)

    pend = (off + pad_cnt).astype(jnp.int32)
    bstart = lax.broadcasted_iota(jnp.int32, (NB, E), 0) * BLK
    n_before = jnp.sum((pend <= bstart).astype(jnp.int32), axis=-1, keepdims=True)
    eid_ref[...] = jnp.minimum(n_before, E - 1)


def _ffn_kernel(eid_ref, x_ref, w1_ref, w2_ref, y_ref):
    del eid_ref
    f = pl.program_id(1)
    x = x_ref[...].astype(jnp.bfloat16)
    h = lax.dot_general(x, w1_ref[0].astype(jnp.bfloat16), (((1,), (0,)), ((), ())),
                        preferred_element_type=jnp.float32)
    h = jnp.maximum(h, 0.0).astype(jnp.bfloat16)
    part = lax.dot_general(h, w2_ref[0].astype(jnp.bfloat16), (((1,), (0,)), ((), ())),
                           preferred_element_type=jnp.float32)

    @pl.when(f == 0)
    def _():
        y_ref[...] = part

    @pl.when(f > 0)
    def _():
        y_ref[...] += part


def _combine_kernel(y0_ref, y1_ref, g0_ref, g1_ref, out_ref):
    out_ref[...] = g0_ref[...] * y0_ref[...] + g1_ref[...] * y1_ref[...]


def _sc_mesh():
    return plsc.VectorSubcoreMesh(core_axis_name="c", subcore_axis_name="s",
                                  num_cores=NC, num_subcores=NS)


def _dispatch_body(flat_hbm, d0_hbm, d1_hbm, xs_hbm, x_v, i0_v, i1_v, s0, s1):
    wid = lax.axis_index("s") * NC + lax.axis_index("c")
    base = wid * TPW
    pltpu.sync_copy(flat_hbm.at[pl.ds(base, TPW)], x_v)
    pltpu.sync_copy(d0_hbm.at[pl.ds(base, TPW)], i0_v)
    pltpu.sync_copy(d1_hbm.at[pl.ds(base, TPW)], i1_v)
    cp0 = pltpu.async_copy(x_v, xs_hbm.at[i0_v], s0)
    cp1 = pltpu.async_copy(x_v, xs_hbm.at[i1_v], s1)
    cp0.wait()
    cp1.wait()


def _sc_dispatch(flat, d0, d1):
    k = pl.kernel(
        _dispatch_body,
        out_type=jax.ShapeDtypeStruct((P, D_MODEL), jnp.float32),
        mesh=_sc_mesh(),
        scratch_types=[
            pltpu.VMEM((TPW, D_MODEL), jnp.float32),
            pltpu.VMEM((TPW,), jnp.int32),
            pltpu.VMEM((TPW,), jnp.int32),
            pltpu.SemaphoreType.DMA,
            pltpu.SemaphoreType.DMA,
        ],
    )
    return k(flat, d0, d1)


def _combine_body(ys_hbm, d0_hbm, d1_hbm, y0_hbm, y1_hbm, rows_v, idx_v, sem):
    wid = lax.axis_index("s") * NC + lax.axis_index("c")
    base = wid * TPW
    for c in range(TPW // CH):
        off = base + c * CH
        pltpu.sync_copy(d0_hbm.at[pl.ds(off, CH)], idx_v)
        pltpu.async_copy(ys_hbm.at[idx_v], rows_v, sem).wait()
        pltpu.sync_copy(rows_v, y0_hbm.at[pl.ds(off, CH)])
        pltpu.sync_copy(d1_hbm.at[pl.ds(off, CH)], idx_v)
        pltpu.async_copy(ys_hbm.at[idx_v], rows_v, sem).wait()
        pltpu.sync_copy(rows_v, y1_hbm.at[pl.ds(off, CH)])


def _sc_combine(ys, d0, d1):
    k = pl.kernel(
        _combine_body,
        out_type=(jax.ShapeDtypeStruct((T, D_MODEL), jnp.float32),
                  jax.ShapeDtypeStruct((T, D_MODEL), jnp.float32)),
        mesh=_sc_mesh(),
        scratch_types=[
            pltpu.VMEM((CH, D_MODEL), jnp.float32),
            pltpu.VMEM((CH,), jnp.int32),
            pltpu.SemaphoreType.DMA,
        ],
    )
    return k(ys, d0, d1)


def kernel(hidden_states, router_w, w1, w2):
    b, s, d = hidden_states.shape
    flat = hidden_states.reshape(T, d)

    d0, d1, g0, g1, eid = pl.pallas_call(
        _route_kernel,
        in_specs=[
            pl.BlockSpec((T, d), lambda: (0, 0)),
            pl.BlockSpec((d, E), lambda: (0, 0)),
        ],
        out_specs=[
            pl.BlockSpec((T, 1), lambda: (0, 0)),
            pl.BlockSpec((T, 1), lambda: (0, 0)),
            pl.BlockSpec((T, 1), lambda: (0, 0)),
            pl.BlockSpec((T, 1), lambda: (0, 0)),
            pl.BlockSpec((NB, 1), lambda: (0, 0)),
        ],
        out_shape=[
            jax.ShapeDtypeStruct((T, 1), jnp.int32),
            jax.ShapeDtypeStruct((T, 1), jnp.int32),
            jax.ShapeDtypeStruct((T, 1), jnp.float32),
            jax.ShapeDtypeStruct((T, 1), jnp.float32),
            jax.ShapeDtypeStruct((NB, 1), jnp.int32),
        ],
    )(flat, router_w)

    d0f = d0.reshape(T)
    d1f = d1.reshape(T)
    eidf = eid.reshape(NB)

    xs = _sc_dispatch(flat, d0f, d1f)

    NF = 2
    F2 = D_FF // NF
    grid_spec = pltpu.PrefetchScalarGridSpec(
        num_scalar_prefetch=1,
        grid=(NB, NF),
        in_specs=[
            pl.BlockSpec((BLK, d), lambda i, f, eid_ref: (i, 0)),
            pl.BlockSpec((1, d, F2), lambda i, f, eid_ref: (eid_ref[i], 0, f)),
            pl.BlockSpec((1, F2, d), lambda i, f, eid_ref: (eid_ref[i], f, 0)),
        ],
        out_specs=pl.BlockSpec((BLK, d), lambda i, f, eid_ref: (i, 0)),
    )
    ys = pl.pallas_call(
        _ffn_kernel,
        grid_spec=grid_spec,
        out_shape=jax.ShapeDtypeStruct((P, d), jnp.float32),
        compiler_params=pltpu.CompilerParams(
            dimension_semantics=("arbitrary", "arbitrary"),
        ),
    )(eidf, xs, w1, w2)

    y0, y1 = _sc_combine(ys, d0f, d1f)

    BT = 512
    out = pl.pallas_call(
        _combine_kernel,
        grid=(T // BT,),
        in_specs=[
            pl.BlockSpec((BT, d), lambda t: (t, 0)),
            pl.BlockSpec((BT, d), lambda t: (t, 0)),
            pl.BlockSpec((BT, 1), lambda t: (t, 0)),
            pl.BlockSpec((BT, 1), lambda t: (t, 0)),
        ],
        out_specs=pl.BlockSpec((BT, d), lambda t: (t, 0)),
        out_shape=jax.ShapeDtypeStruct((T, d), jnp.float32),
    )(y0, y1, g0, g1)

    return out.reshape(b, s, d)


# speedup vs baseline: 1.0046x; 1.0046x over previous
import functools

import jax
import jax.numpy as jnp
from jax import lax
from jax.experimental import pallas as pl
from jax.experimental.pallas import tpu as pltpu
from jax.experimental.pallas import tpu_sc as plsc

E = 8
TOP_K = 2
D_MODEL = 1024
D_FF = 2048
T = 2048
BLK = 256
NB = 24
P = NB * BLK
NEG_INF = -1e30

NC = 2
NS = 16
NW = NC * NS
TPW = T // NW
CH = 32

_HI = jax.lax.Precision.HIGHEST


def _route_kernel(x_ref, rw_ref, d0_ref, d1_ref, g0_ref, g1_ref, eid_ref):
    x = x_ref[...]
    logits = lax.dot_general(x, rw_ref[...], (((1,), (0,)), ((), ())),
                             preferred_element_type=jnp.float32)
    eids = lax.broadcasted_iota(jnp.int32, (T, E), 1)
    m1 = jnp.max(logits, axis=-1, keepdims=True)
    e1 = jnp.min(jnp.where(logits >= m1, eids, E), axis=-1, keepdims=True)
    l2 = jnp.where(eids == e1, NEG_INF, logits)
    m2 = jnp.max(l2, axis=-1, keepdims=True)
    e2 = jnp.min(jnp.where(l2 >= m2, eids, E), axis=-1, keepdims=True)
    r = jnp.exp(m2 - m1)
    g0_ref[...] = 1.0 / (1.0 + r)
    g1_ref[...] = r / (1.0 + r)

    oh0 = (eids == e1).astype(jnp.float32)
    oh1 = (eids == e2).astype(jnp.float32)
    oh = jnp.concatenate([oh0, oh1], axis=1)
    tri = (lax.broadcasted_iota(jnp.int32, (256, 256), 0)
           >= lax.broadcasted_iota(jnp.int32, (256, 256), 1)).astype(jnp.float32)
    parts = []
    carry = jnp.zeros((1, 2 * E), jnp.float32)
    for k in range(T // 256):
        p = lax.dot_general(tri, oh[k * 256:(k + 1) * 256], (((1,), (0,)), ((), ())),
                            precision=_HI, preferred_element_type=jnp.float32) + carry
        parts.append(p)
        carry = p[-1:, :]
    c = jnp.concatenate(parts, axis=0)
    c0 = c[:, :E]
    c1 = c[:, E:]
    c0ex = c0 - oh0
    c1ex = c1 - oh1

    cnt = carry[:, :E] + carry[:, E:]
    pad_cnt = (((cnt.astype(jnp.int32) + (BLK - 1)) >> 8) << 8).astype(jnp.float32)
    m8 = (lax.broadcasted_iota(jnp.int32, (E, E), 0)
          < lax.broadcasted_iota(jnp.int32, (E, E), 1)).astype(jnp.float32)
    off = lax.dot_general(pad_cnt, m8, (((1,), (0,)), ((), ())),
                          precision=_HI, preferred_element_type=jnp.float32)

    rank0 = c0ex + c1ex
    rank1 = c0 + c1ex
    d0_ref[...] = jnp.sum(oh0 * (off + rank0), axis=-1, keepdims=True).astype(jnp.int32)
    d1_ref[...] = jnp.sum(oh1 * (off + rank1), axis=-1, keepdims=True).astype(jnp.int32)

    pend = (off + pad_cnt).astype(jnp.int32)
    bstart = lax.broadcasted_iota(jnp.int32, (NB, E), 0) * BLK
    n_before = jnp.sum((pend <= bstart).astype(jnp.int32), axis=-1, keepdims=True)
    eid_ref[...] = jnp.minimum(n_before, E - 1)


def _ffn_kernel(eid_ref, x_ref, w1_ref, w2_ref, y_ref):
    del eid_ref
    f = pl.program_id(1)
    h = lax.dot_general(x_ref[...], w1_ref[0], (((1,), (0,)), ((), ())),
                        preferred_element_type=jnp.float32)
    h = jnp.maximum(h, 0.0)
    part = lax.dot_general(h, w2_ref[0], (((1,), (0,)), ((), ())),
                           preferred_element_type=jnp.float32)

    @pl.when(f == 0)
    def _():
        y_ref[...] = part

    @pl.when(f > 0)
    def _():
        y_ref[...] += part


def _combine_kernel(y0_ref, y1_ref, g0_ref, g1_ref, out_ref):
    out_ref[...] = g0_ref[...] * y0_ref[...] + g1_ref[...] * y1_ref[...]


def _sc_mesh():
    return plsc.VectorSubcoreMesh(core_axis_name="c", subcore_axis_name="s",
                                  num_cores=NC, num_subcores=NS)


def _dispatch_body(flat_hbm, d0_hbm, d1_hbm, xs_hbm, x_v, i0_v, i1_v, s0, s1):
    wid = lax.axis_index("s") * NC + lax.axis_index("c")
    base = wid * TPW
    pltpu.sync_copy(flat_hbm.at[pl.ds(base, TPW)], x_v)
    pltpu.sync_copy(d0_hbm.at[pl.ds(base, TPW)], i0_v)
    pltpu.sync_copy(d1_hbm.at[pl.ds(base, TPW)], i1_v)
    cp0 = pltpu.async_copy(x_v, xs_hbm.at[i0_v], s0)
    cp1 = pltpu.async_copy(x_v, xs_hbm.at[i1_v], s1)
    cp0.wait()
    cp1.wait()


def _sc_dispatch(flat, d0, d1):
    k = pl.kernel(
        _dispatch_body,
        out_type=jax.ShapeDtypeStruct((P, D_MODEL), jnp.float32),
        mesh=_sc_mesh(),
        scratch_types=[
            pltpu.VMEM((TPW, D_MODEL), jnp.float32),
            pltpu.VMEM((TPW,), jnp.int32),
            pltpu.VMEM((TPW,), jnp.int32),
            pltpu.SemaphoreType.DMA,
            pltpu.SemaphoreType.DMA,
        ],
    )
    return k(flat, d0, d1)


def _combine_body(ys_hbm, d0_hbm, d1_hbm, y0_hbm, y1_hbm, rows_v, idx_v, sem):
    wid = lax.axis_index("s") * NC + lax.axis_index("c")
    base = wid * TPW
    for c in range(TPW // CH):
        off = base + c * CH
        pltpu.sync_copy(d0_hbm.at[pl.ds(off, CH)], idx_v)
        pltpu.async_copy(ys_hbm.at[idx_v], rows_v, sem).wait()
        pltpu.sync_copy(rows_v, y0_hbm.at[pl.ds(off, CH)])
        pltpu.sync_copy(d1_hbm.at[pl.ds(off, CH)], idx_v)
        pltpu.async_copy(ys_hbm.at[idx_v], rows_v, sem).wait()
        pltpu.sync_copy(rows_v, y1_hbm.at[pl.ds(off, CH)])


def _sc_combine(ys, d0, d1):
    k = pl.kernel(
        _combine_body,
        out_type=(jax.ShapeDtypeStruct((T, D_MODEL), jnp.float32),
                  jax.ShapeDtypeStruct((T, D_MODEL), jnp.float32)),
        mesh=_sc_mesh(),
        scratch_types=[
            pltpu.VMEM((CH, D_MODEL), jnp.float32),
            pltpu.VMEM((CH,), jnp.int32),
            pltpu.SemaphoreType.DMA,
        ],
    )
    return k(ys, d0, d1)


def kernel(hidden_states, router_w, w1, w2):
    b, s, d = hidden_states.shape
    flat = hidden_states.reshape(T, d)

    d0, d1, g0, g1, eid = pl.pallas_call(
        _route_kernel,
        in_specs=[
            pl.BlockSpec((T, d), lambda: (0, 0)),
            pl.BlockSpec((d, E), lambda: (0, 0)),
        ],
        out_specs=[
            pl.BlockSpec((T, 1), lambda: (0, 0)),
            pl.BlockSpec((T, 1), lambda: (0, 0)),
            pl.BlockSpec((T, 1), lambda: (0, 0)),
            pl.BlockSpec((T, 1), lambda: (0, 0)),
            pl.BlockSpec((NB, 1), lambda: (0, 0)),
        ],
        out_shape=[
            jax.ShapeDtypeStruct((T, 1), jnp.int32),
            jax.ShapeDtypeStruct((T, 1), jnp.int32),
            jax.ShapeDtypeStruct((T, 1), jnp.float32),
            jax.ShapeDtypeStruct((T, 1), jnp.float32),
            jax.ShapeDtypeStruct((NB, 1), jnp.int32),
        ],
    )(flat, router_w)

    d0f = d0.reshape(T)
    d1f = d1.reshape(T)
    eidf = eid.reshape(NB)

    xs = _sc_dispatch(flat, d0f, d1f)

    NF = 2
    F2 = D_FF // NF
    grid_spec = pltpu.PrefetchScalarGridSpec(
        num_scalar_prefetch=1,
        grid=(NB, NF),
        in_specs=[
            pl.BlockSpec((BLK, d), lambda i, f, eid_ref: (i, 0)),
            pl.BlockSpec((1, d, F2), lambda i, f, eid_ref: (eid_ref[i], 0, f)),
            pl.BlockSpec((1, F2, d), lambda i, f, eid_ref: (eid_ref[i], f, 0)),
        ],
        out_specs=pl.BlockSpec((BLK, d), lambda i, f, eid_ref: (i, 0)),
    )
    ys = pl.pallas_call(
        _ffn_kernel,
        grid_spec=grid_spec,
        out_shape=jax.ShapeDtypeStruct((P, d), jnp.float32),
        compiler_params=pltpu.CompilerParams(
            dimension_semantics=("arbitrary", "arbitrary"),
        ),
    )(eidf, xs, w1, w2)

    y0, y1 = _sc_combine(ys, d0f, d1f)

    BT = 512
    out = pl.pallas_call(
        _combine_kernel,
        grid=(T // BT,),
        in_specs=[
            pl.BlockSpec((BT, d), lambda t: (t, 0)),
            pl.BlockSpec((BT, d), lambda t: (t, 0)),
            pl.BlockSpec((BT, 1), lambda t: (t, 0)),
            pl.BlockSpec((BT, 1), lambda t: (t, 0)),
        ],
        out_specs=pl.BlockSpec((BT, d), lambda t: (t, 0)),
        out_shape=jax.ShapeDtypeStruct((T, d), jnp.float32),
    )(y0, y1, g0, g1)

    return out.reshape(b, s, d)


# speedup vs baseline: 1.1483x; 1.1430x over previous
import functools

import jax
import jax.numpy as jnp
from jax import lax
from jax.experimental import pallas as pl
from jax.experimental.pallas import tpu as pltpu
from jax.experimental.pallas import tpu_sc as plsc

E = 8
TOP_K = 2
D_MODEL = 1024
D_FF = 2048
T = 2048
BLK = 256
NB = 24
JMAX = T // BLK
P = (NB + 1) * BLK
NEG_INF = -1e30

NC = 2
NS = 16
NW = NC * NS
TPW = T // NW
CH = 32

_HI = jax.lax.Precision.HIGHEST


def _route_kernel(x_ref, rw_ref, d0_ref, d1_ref, g0_ref, g1_ref, bi_ref, live_ref):
    x = x_ref[...]
    logits = lax.dot_general(x, rw_ref[...], (((1,), (0,)), ((), ())),
                             preferred_element_type=jnp.float32)
    eids = lax.broadcasted_iota(jnp.int32, (T, E), 1)
    m1 = jnp.max(logits, axis=-1, keepdims=True)
    e1 = jnp.min(jnp.where(logits >= m1, eids, E), axis=-1, keepdims=True)
    l2 = jnp.where(eids == e1, NEG_INF, logits)
    m2 = jnp.max(l2, axis=-1, keepdims=True)
    e2 = jnp.min(jnp.where(l2 >= m2, eids, E), axis=-1, keepdims=True)
    r = jnp.exp(m2 - m1)
    g0_ref[...] = 1.0 / (1.0 + r)
    g1_ref[...] = r / (1.0 + r)

    oh0 = (eids == e1).astype(jnp.float32)
    oh1 = (eids == e2).astype(jnp.float32)
    oh = jnp.concatenate([oh0, oh1], axis=1)
    tri = (lax.broadcasted_iota(jnp.int32, (256, 256), 0)
           >= lax.broadcasted_iota(jnp.int32, (256, 256), 1)).astype(jnp.float32)
    parts = []
    carry = jnp.zeros((1, 2 * E), jnp.float32)
    for k in range(T // 256):
        p = lax.dot_general(tri, oh[k * 256:(k + 1) * 256], (((1,), (0,)), ((), ())),
                            precision=_HI, preferred_element_type=jnp.float32) + carry
        parts.append(p)
        carry = p[-1:, :]
    c = jnp.concatenate(parts, axis=0)
    c0 = c[:, :E]
    c1 = c[:, E:]
    c0ex = c0 - oh0
    c1ex = c1 - oh1

    cnt = carry[:, :E] + carry[:, E:]
    pad_cnt = (((cnt.astype(jnp.int32) + (BLK - 1)) >> 8) << 8).astype(jnp.float32)
    m8 = (lax.broadcasted_iota(jnp.int32, (E, E), 0)
          < lax.broadcasted_iota(jnp.int32, (E, E), 1)).astype(jnp.float32)
    off = lax.dot_general(pad_cnt, m8, (((1,), (0,)), ((), ())),
                          precision=_HI, preferred_element_type=jnp.float32)

    rank0 = c0ex + c1ex
    rank1 = c0 + c1ex
    d0_ref[...] = jnp.sum(oh0 * (off + rank0), axis=-1, keepdims=True).astype(jnp.int32)
    d1_ref[...] = jnp.sum(oh1 * (off + rank1), axis=-1, keepdims=True).astype(jnp.int32)

    i8 = (lax.broadcasted_iota(jnp.int32, (E, E), 0)
          == lax.broadcasted_iota(jnp.int32, (E, E), 1)).astype(jnp.float32)
    pad_cnt_col = lax.dot_general(i8, pad_cnt, (((1,), (1,)), ((), ())),
                                  precision=_HI, preferred_element_type=jnp.float32)
    tri8 = (lax.broadcasted_iota(jnp.int32, (E, E), 1)
            < lax.broadcasted_iota(jnp.int32, (E, E), 0)).astype(jnp.float32)
    base_col = lax.dot_general(tri8, pad_cnt_col, (((1,), (0,)), ((), ())),
                               precision=_HI, preferred_element_type=jnp.float32)
    nblk_col = pad_cnt_col.astype(jnp.int32) >> 8
    base_blk_col = base_col.astype(jnp.int32) >> 8
    jio = lax.broadcasted_iota(jnp.int32, (E, JMAX), 1)
    live = (jio < nblk_col).astype(jnp.int32)
    live_ref[...] = live
    bi_ref[...] = jnp.where(live == 1, base_blk_col + jio, NB)


def _ffn_kernel(bi_ref, live_ref, x_ref, w1_ref, w2_ref, y_ref):
    e = pl.program_id(0)
    j = pl.program_id(1)

    @pl.when(live_ref[e, j] == 1)
    def _():
        h = lax.dot_general(x_ref[...], w1_ref[0], (((1,), (0,)), ((), ())),
                            preferred_element_type=jnp.float32)
        h = jnp.maximum(h, 0.0)
        y_ref[...] = lax.dot_general(h, w2_ref[0], (((1,), (0,)), ((), ())),
                                     preferred_element_type=jnp.float32)


def _combine_kernel(y0_ref, y1_ref, g0_ref, g1_ref, out_ref):
    out_ref[...] = g0_ref[...] * y0_ref[...] + g1_ref[...] * y1_ref[...]


def _sc_mesh():
    return plsc.VectorSubcoreMesh(core_axis_name="c", subcore_axis_name="s",
                                  num_cores=NC, num_subcores=NS)


def _dispatch_body(flat_hbm, d0_hbm, d1_hbm, xs_hbm, x_v, i0_v, i1_v, s0, s1):
    wid = lax.axis_index("s") * NC + lax.axis_index("c")
    base = wid * TPW
    pltpu.sync_copy(flat_hbm.at[pl.ds(base, TPW)], x_v)
    pltpu.sync_copy(d0_hbm.at[pl.ds(base, TPW)], i0_v)
    pltpu.sync_copy(d1_hbm.at[pl.ds(base, TPW)], i1_v)
    cp0 = pltpu.async_copy(x_v, xs_hbm.at[i0_v], s0)
    cp1 = pltpu.async_copy(x_v, xs_hbm.at[i1_v], s1)
    cp0.wait()
    cp1.wait()


def _sc_dispatch(flat, d0, d1):
    k = pl.kernel(
        _dispatch_body,
        out_type=jax.ShapeDtypeStruct((P, D_MODEL), jnp.float32),
        mesh=_sc_mesh(),
        scratch_types=[
            pltpu.VMEM((TPW, D_MODEL), jnp.float32),
            pltpu.VMEM((TPW,), jnp.int32),
            pltpu.VMEM((TPW,), jnp.int32),
            pltpu.SemaphoreType.DMA,
            pltpu.SemaphoreType.DMA,
        ],
    )
    return k(flat, d0, d1)


def _combine_body(ys_hbm, d0_hbm, d1_hbm, y0_hbm, y1_hbm, rows_v, idx_v, sem):
    wid = lax.axis_index("s") * NC + lax.axis_index("c")
    base = wid * TPW
    for c in range(TPW // CH):
        off = base + c * CH
        pltpu.sync_copy(d0_hbm.at[pl.ds(off, CH)], idx_v)
        pltpu.async_copy(ys_hbm.at[idx_v], rows_v, sem).wait()
        pltpu.sync_copy(rows_v, y0_hbm.at[pl.ds(off, CH)])
        pltpu.sync_copy(d1_hbm.at[pl.ds(off, CH)], idx_v)
        pltpu.async_copy(ys_hbm.at[idx_v], rows_v, sem).wait()
        pltpu.sync_copy(rows_v, y1_hbm.at[pl.ds(off, CH)])


def _sc_combine(ys, d0, d1):
    k = pl.kernel(
        _combine_body,
        out_type=(jax.ShapeDtypeStruct((T, D_MODEL), jnp.float32),
                  jax.ShapeDtypeStruct((T, D_MODEL), jnp.float32)),
        mesh=_sc_mesh(),
        scratch_types=[
            pltpu.VMEM((CH, D_MODEL), jnp.float32),
            pltpu.VMEM((CH,), jnp.int32),
            pltpu.SemaphoreType.DMA,
        ],
    )
    return k(ys, d0, d1)


def kernel(hidden_states, router_w, w1, w2):
    b, s, d = hidden_states.shape
    flat = hidden_states.reshape(T, d)

    d0, d1, g0, g1, bi, live = pl.pallas_call(
        _route_kernel,
        in_specs=[
            pl.BlockSpec((T, d), lambda: (0, 0)),
            pl.BlockSpec((d, E), lambda: (0, 0)),
        ],
        out_specs=[
            pl.BlockSpec((T, 1), lambda: (0, 0)),
            pl.BlockSpec((T, 1), lambda: (0, 0)),
            pl.BlockSpec((T, 1), lambda: (0, 0)),
            pl.BlockSpec((T, 1), lambda: (0, 0)),
            pl.BlockSpec((E, JMAX), lambda: (0, 0)),
            pl.BlockSpec((E, JMAX), lambda: (0, 0)),
        ],
        out_shape=[
            jax.ShapeDtypeStruct((T, 1), jnp.int32),
            jax.ShapeDtypeStruct((T, 1), jnp.int32),
            jax.ShapeDtypeStruct((T, 1), jnp.float32),
            jax.ShapeDtypeStruct((T, 1), jnp.float32),
            jax.ShapeDtypeStruct((E, JMAX), jnp.int32),
            jax.ShapeDtypeStruct((E, JMAX), jnp.int32),
        ],
    )(flat, router_w)

    d0f = d0.reshape(T)
    d1f = d1.reshape(T)

    xs = _sc_dispatch(flat, d0f, d1f)

    grid_spec = pltpu.PrefetchScalarGridSpec(
        num_scalar_prefetch=2,
        grid=(E, JMAX),
        in_specs=[
            pl.BlockSpec((BLK, d), lambda e, j, bi_r, lv_r: (bi_r[e, j], 0)),
            pl.BlockSpec((1, d, D_FF), lambda e, j, bi_r, lv_r: (e, 0, 0)),
            pl.BlockSpec((1, D_FF, d), lambda e, j, bi_r, lv_r: (e, 0, 0)),
        ],
        out_specs=pl.BlockSpec((BLK, d), lambda e, j, bi_r, lv_r: (bi_r[e, j], 0)),
    )
    ys = pl.pallas_call(
        _ffn_kernel,
        grid_spec=grid_spec,
        out_shape=jax.ShapeDtypeStruct((P, d), jnp.float32),
        compiler_params=pltpu.CompilerParams(
            dimension_semantics=("arbitrary", "arbitrary"),
        ),
    )(bi, live, xs, w1, w2)

    y0, y1 = _sc_combine(ys, d0f, d1f)

    BT = 512
    out = pl.pallas_call(
        _combine_kernel,
        grid=(T // BT,),
        in_specs=[
            pl.BlockSpec((BT, d), lambda t: (t, 0)),
            pl.BlockSpec((BT, d), lambda t: (t, 0)),
            pl.BlockSpec((BT, 1), lambda t: (t, 0)),
            pl.BlockSpec((BT, 1), lambda t: (t, 0)),
        ],
        out_specs=pl.BlockSpec((BT, d), lambda t: (t, 0)),
        out_shape=jax.ShapeDtypeStruct((T, d), jnp.float32),
    )(y0, y1, g0, g1)

    return out.reshape(b, s, d)


# speedup vs baseline: 1.2818x; 1.1162x over previous
import functools

import jax
import jax.numpy as jnp
from jax import lax
from jax.experimental import pallas as pl
from jax.experimental.pallas import tpu as pltpu
from jax.experimental.pallas import tpu_sc as plsc

E = 8
TOP_K = 2
D_MODEL = 1024
D_FF = 2048
T = 2048
BLK = 256
NB = 24
P = NB * BLK
NEG_INF = -1e30

NC = 2
NS = 16
NW = NC * NS
TPW = T // NW
CH = 32

_HI = jax.lax.Precision.HIGHEST


def _route_kernel(x_ref, rw_ref, d0_ref, d1_ref, g0_ref, g1_ref, eid_ref, lb_ref):
    x = x_ref[...]
    logits = lax.dot_general(x, rw_ref[...], (((1,), (0,)), ((), ())),
                             preferred_element_type=jnp.float32)
    eids = lax.broadcasted_iota(jnp.int32, (T, E), 1)
    m1 = jnp.max(logits, axis=-1, keepdims=True)
    e1 = jnp.min(jnp.where(logits >= m1, eids, E), axis=-1, keepdims=True)
    l2 = jnp.where(eids == e1, NEG_INF, logits)
    m2 = jnp.max(l2, axis=-1, keepdims=True)
    e2 = jnp.min(jnp.where(l2 >= m2, eids, E), axis=-1, keepdims=True)
    r = jnp.exp(m2 - m1)
    g0_ref[...] = 1.0 / (1.0 + r)
    g1_ref[...] = r / (1.0 + r)

    oh0 = (eids == e1).astype(jnp.float32)
    oh1 = (eids == e2).astype(jnp.float32)
    oh = jnp.concatenate([oh0, oh1], axis=1)
    tri = (lax.broadcasted_iota(jnp.int32, (256, 256), 0)
           >= lax.broadcasted_iota(jnp.int32, (256, 256), 1)).astype(jnp.float32)
    parts = []
    carry = jnp.zeros((1, 2 * E), jnp.float32)
    for k in range(T // 256):
        p = lax.dot_general(tri, oh[k * 256:(k + 1) * 256], (((1,), (0,)), ((), ())),
                            precision=_HI, preferred_element_type=jnp.float32) + carry
        parts.append(p)
        carry = p[-1:, :]
    c = jnp.concatenate(parts, axis=0)
    c0 = c[:, :E]
    c1 = c[:, E:]
    c0ex = c0 - oh0
    c1ex = c1 - oh1

    cnt = carry[:, :E] + carry[:, E:]
    pad_cnt = (((cnt.astype(jnp.int32) + (BLK - 1)) >> 8) << 8).astype(jnp.float32)
    m8 = (lax.broadcasted_iota(jnp.int32, (E, E), 0)
          < lax.broadcasted_iota(jnp.int32, (E, E), 1)).astype(jnp.float32)
    off = lax.dot_general(pad_cnt, m8, (((1,), (0,)), ((), ())),
                          precision=_HI, preferred_element_type=jnp.float32)

    rank0 = c0ex + c1ex
    rank1 = c0 + c1ex
    d0_ref[...] = jnp.sum(oh0 * (off + rank0), axis=-1, keepdims=True).astype(jnp.int32)
    d1_ref[...] = jnp.sum(oh1 * (off + rank1), axis=-1, keepdims=True).astype(jnp.int32)

    pend = (off + pad_cnt).astype(jnp.int32)
    bstart = lax.broadcasted_iota(jnp.int32, (NB, E), 0) * BLK
    n_before = jnp.sum((pend <= bstart).astype(jnp.int32), axis=-1, keepdims=True)
    eid_ref[...] = jnp.minimum(n_before, E - 1)
    total = jnp.sum(pad_cnt, axis=-1, keepdims=True).astype(jnp.int32)
    bstart1 = lax.broadcasted_iota(jnp.int32, (NB, 1), 0) * BLK
    lb_ref[...] = (bstart1 < total).astype(jnp.int32)


def _ffn_kernel(eid_ref, lb_ref, x_ref, w1_hbm, w2_hbm, y_ref,
                w1a, w2a, w1b, w2b, slot_ref, s1a, s2a, s1b, s2b):
    i = pl.program_id(0)
    e_cur = eid_ref[i, 0]
    e_prv = eid_ref[jnp.maximum(i - 1, 0), 0]
    prev_s = slot_ref[0]
    s = jnp.where(i == 0, 0, jnp.where(e_cur != e_prv, 1 - prev_s, prev_s))
    slot_ref[0] = s
    chg = (i == 0) | (e_cur != e_prv)
    e_nxt = eid_ref[jnp.minimum(i + 1, NB - 1), 0]
    pref = (i + 1 < NB) & (e_nxt != e_cur)

    @pl.when(i == 0)
    def _():
        pltpu.make_async_copy(w1_hbm.at[e_cur], w1a, s1a).start()
        pltpu.make_async_copy(w2_hbm.at[e_cur], w2a, s2a).start()

    @pl.when(chg & (s == 0))
    def _():
        pltpu.make_async_copy(w1_hbm.at[e_cur], w1a, s1a).wait()
        pltpu.make_async_copy(w2_hbm.at[e_cur], w2a, s2a).wait()

    @pl.when(chg & (s == 1))
    def _():
        pltpu.make_async_copy(w1_hbm.at[e_cur], w1b, s1b).wait()
        pltpu.make_async_copy(w2_hbm.at[e_cur], w2b, s2b).wait()

    @pl.when(pref & (s == 0))
    def _():
        pltpu.make_async_copy(w1_hbm.at[e_nxt], w1b, s1b).start()
        pltpu.make_async_copy(w2_hbm.at[e_nxt], w2b, s2b).start()

    @pl.when(pref & (s == 1))
    def _():
        pltpu.make_async_copy(w1_hbm.at[e_nxt], w1a, s1a).start()
        pltpu.make_async_copy(w2_hbm.at[e_nxt], w2a, s2a).start()

    live = lb_ref[i, 0] == 1

    @pl.when(live & (s == 0))
    def _():
        h = lax.dot_general(x_ref[...], w1a[...], (((1,), (0,)), ((), ())),
                            preferred_element_type=jnp.float32)
        h = jnp.maximum(h, 0.0)
        y_ref[...] = lax.dot_general(h, w2a[...], (((1,), (0,)), ((), ())),
                                     preferred_element_type=jnp.float32)

    @pl.when(live & (s == 1))
    def _():
        h = lax.dot_general(x_ref[...], w1b[...], (((1,), (0,)), ((), ())),
                            preferred_element_type=jnp.float32)
        h = jnp.maximum(h, 0.0)
        y_ref[...] = lax.dot_general(h, w2b[...], (((1,), (0,)), ((), ())),
                                     preferred_element_type=jnp.float32)


def _combine_kernel(y0_ref, y1_ref, g0_ref, g1_ref, out_ref):
    out_ref[...] = g0_ref[...] * y0_ref[...] + g1_ref[...] * y1_ref[...]


def _sc_mesh():
    return plsc.VectorSubcoreMesh(core_axis_name="c", subcore_axis_name="s",
                                  num_cores=NC, num_subcores=NS)


def _dispatch_body(flat_hbm, d0_hbm, d1_hbm, xs_hbm, x_v, i0_v, i1_v, s0, s1):
    wid = lax.axis_index("s") * NC + lax.axis_index("c")
    base = wid * TPW
    pltpu.sync_copy(flat_hbm.at[pl.ds(base, TPW)], x_v)
    pltpu.sync_copy(d0_hbm.at[pl.ds(base, TPW)], i0_v)
    pltpu.sync_copy(d1_hbm.at[pl.ds(base, TPW)], i1_v)
    cp0 = pltpu.async_copy(x_v, xs_hbm.at[i0_v], s0)
    cp1 = pltpu.async_copy(x_v, xs_hbm.at[i1_v], s1)
    cp0.wait()
    cp1.wait()


def _sc_dispatch(flat, d0, d1):
    k = pl.kernel(
        _dispatch_body,
        out_type=jax.ShapeDtypeStruct((P, D_MODEL), jnp.float32),
        mesh=_sc_mesh(),
        scratch_types=[
            pltpu.VMEM((TPW, D_MODEL), jnp.float32),
            pltpu.VMEM((TPW,), jnp.int32),
            pltpu.VMEM((TPW,), jnp.int32),
            pltpu.SemaphoreType.DMA,
            pltpu.SemaphoreType.DMA,
        ],
    )
    return k(flat, d0, d1)


def _combine_body(ys_hbm, d0_hbm, d1_hbm, y0_hbm, y1_hbm, rows_v, idx_v, sem):
    wid = lax.axis_index("s") * NC + lax.axis_index("c")
    base = wid * TPW
    for c in range(TPW // CH):
        off = base + c * CH
        pltpu.sync_copy(d0_hbm.at[pl.ds(off, CH)], idx_v)
        pltpu.async_copy(ys_hbm.at[idx_v], rows_v, sem).wait()
        pltpu.sync_copy(rows_v, y0_hbm.at[pl.ds(off, CH)])
        pltpu.sync_copy(d1_hbm.at[pl.ds(off, CH)], idx_v)
        pltpu.async_copy(ys_hbm.at[idx_v], rows_v, sem).wait()
        pltpu.sync_copy(rows_v, y1_hbm.at[pl.ds(off, CH)])


def _sc_combine(ys, d0, d1):
    k = pl.kernel(
        _combine_body,
        out_type=(jax.ShapeDtypeStruct((T, D_MODEL), jnp.float32),
                  jax.ShapeDtypeStruct((T, D_MODEL), jnp.float32)),
        mesh=_sc_mesh(),
        scratch_types=[
            pltpu.VMEM((CH, D_MODEL), jnp.float32),
            pltpu.VMEM((CH,), jnp.int32),
            pltpu.SemaphoreType.DMA,
        ],
    )
    return k(ys, d0, d1)


def kernel(hidden_states, router_w, w1, w2):
    b, s, d = hidden_states.shape
    flat = hidden_states.reshape(T, d)

    d0, d1, g0, g1, eid, lb = pl.pallas_call(
        _route_kernel,
        in_specs=[
            pl.BlockSpec((T, d), lambda: (0, 0)),
            pl.BlockSpec((d, E), lambda: (0, 0)),
        ],
        out_specs=[
            pl.BlockSpec((T, 1), lambda: (0, 0)),
            pl.BlockSpec((T, 1), lambda: (0, 0)),
            pl.BlockSpec((T, 1), lambda: (0, 0)),
            pl.BlockSpec((T, 1), lambda: (0, 0)),
            pl.BlockSpec((NB, 1), lambda: (0, 0)),
            pl.BlockSpec((NB, 1), lambda: (0, 0)),
        ],
        out_shape=[
            jax.ShapeDtypeStruct((T, 1), jnp.int32),
            jax.ShapeDtypeStruct((T, 1), jnp.int32),
            jax.ShapeDtypeStruct((T, 1), jnp.float32),
            jax.ShapeDtypeStruct((T, 1), jnp.float32),
            jax.ShapeDtypeStruct((NB, 1), jnp.int32),
            jax.ShapeDtypeStruct((NB, 1), jnp.int32),
        ],
    )(flat, router_w)

    d0f = d0.reshape(T)
    d1f = d1.reshape(T)

    xs = _sc_dispatch(flat, d0f, d1f)

    grid_spec = pltpu.PrefetchScalarGridSpec(
        num_scalar_prefetch=2,
        grid=(NB,),
        in_specs=[
            pl.BlockSpec((BLK, d), lambda i, eid_r, lb_r: (i, 0)),
            pl.BlockSpec(memory_space=pl.ANY),
            pl.BlockSpec(memory_space=pl.ANY),
        ],
        out_specs=pl.BlockSpec((BLK, d), lambda i, eid_r, lb_r: (i, 0)),
        scratch_shapes=[
            pltpu.VMEM((D_MODEL, D_FF), jnp.float32),
            pltpu.VMEM((D_FF, D_MODEL), jnp.float32),
            pltpu.VMEM((D_MODEL, D_FF), jnp.float32),
            pltpu.VMEM((D_FF, D_MODEL), jnp.float32),
            pltpu.SMEM((1,), jnp.int32),
            pltpu.SemaphoreType.DMA,
            pltpu.SemaphoreType.DMA,
            pltpu.SemaphoreType.DMA,
            pltpu.SemaphoreType.DMA,
        ],
    )
    ys = pl.pallas_call(
        _ffn_kernel,
        grid_spec=grid_spec,
        out_shape=jax.ShapeDtypeStruct((P, d), jnp.float32),
        compiler_params=pltpu.CompilerParams(
            dimension_semantics=("arbitrary",),
        ),
    )(eid, lb, xs, w1, w2)

    y0, y1 = _sc_combine(ys, d0f, d1f)

    BT = 512
    out = pl.pallas_call(
        _combine_kernel,
        grid=(T // BT,),
        in_specs=[
            pl.BlockSpec((BT, d), lambda t: (t, 0)),
            pl.BlockSpec((BT, d), lambda t: (t, 0)),
            pl.BlockSpec((BT, 1), lambda t: (t, 0)),
            pl.BlockSpec((BT, 1), lambda t: (t, 0)),
        ],
        out_specs=pl.BlockSpec((BT, d), lambda t: (t, 0)),
        out_shape=jax.ShapeDtypeStruct((T, d), jnp.float32),
    )(y0, y1, g0, g1)

    return out.reshape(b, s, d)


# speedup vs baseline: 1.4213x; 1.1088x over previous
import functools

import jax
import jax.numpy as jnp
from jax import lax
from jax.experimental import pallas as pl
from jax.experimental.pallas import tpu as pltpu
from jax.experimental.pallas import tpu_sc as plsc

E = 8
TOP_K = 2
D_MODEL = 1024
D_FF = 2048
T = 2048
BLK = 256
NB = 24
P = NB * BLK
NEG_INF = -1e30

NC = 2
NS = 16
NW = NC * NS
TPW = T // NW
CH = 32

_HI = jax.lax.Precision.HIGHEST


def _route_kernel(x_ref, rw_ref, d0_ref, d1_ref, g0_ref, g1_ref, eid_ref, lb_ref,
                  nxt_ref):
    x = x_ref[...]
    logits = lax.dot_general(x, rw_ref[...], (((1,), (0,)), ((), ())),
                             preferred_element_type=jnp.float32)
    eids = lax.broadcasted_iota(jnp.int32, (T, E), 1)
    m1 = jnp.max(logits, axis=-1, keepdims=True)
    e1 = jnp.min(jnp.where(logits >= m1, eids, E), axis=-1, keepdims=True)
    l2 = jnp.where(eids == e1, NEG_INF, logits)
    m2 = jnp.max(l2, axis=-1, keepdims=True)
    e2 = jnp.min(jnp.where(l2 >= m2, eids, E), axis=-1, keepdims=True)
    r = jnp.exp(m2 - m1)
    g0_ref[...] = 1.0 / (1.0 + r)
    g1_ref[...] = r / (1.0 + r)

    oh0 = (eids == e1).astype(jnp.float32)
    oh1 = (eids == e2).astype(jnp.float32)
    oh = jnp.concatenate([oh0, oh1], axis=1)
    tri = (lax.broadcasted_iota(jnp.int32, (256, 256), 0)
           >= lax.broadcasted_iota(jnp.int32, (256, 256), 1)).astype(jnp.float32)
    parts = []
    carry = jnp.zeros((1, 2 * E), jnp.float32)
    for k in range(T // 256):
        p = lax.dot_general(tri, oh[k * 256:(k + 1) * 256], (((1,), (0,)), ((), ())),
                            precision=_HI, preferred_element_type=jnp.float32) + carry
        parts.append(p)
        carry = p[-1:, :]
    c = jnp.concatenate(parts, axis=0)
    c0 = c[:, :E]
    c1 = c[:, E:]
    c0ex = c0 - oh0
    c1ex = c1 - oh1

    cnt = carry[:, :E] + carry[:, E:]
    pad_cnt = (((cnt.astype(jnp.int32) + (BLK - 1)) >> 8) << 8).astype(jnp.float32)
    m8 = (lax.broadcasted_iota(jnp.int32, (E, E), 0)
          < lax.broadcasted_iota(jnp.int32, (E, E), 1)).astype(jnp.float32)
    off = lax.dot_general(pad_cnt, m8, (((1,), (0,)), ((), ())),
                          precision=_HI, preferred_element_type=jnp.float32)

    rank0 = c0ex + c1ex
    rank1 = c0 + c1ex
    d0_ref[...] = jnp.sum(oh0 * (off + rank0), axis=-1, keepdims=True).astype(jnp.int32)
    d1_ref[...] = jnp.sum(oh1 * (off + rank1), axis=-1, keepdims=True).astype(jnp.int32)

    pend = (off + pad_cnt).astype(jnp.int32)
    bstart = lax.broadcasted_iota(jnp.int32, (NB, E), 0) * BLK
    n_before = jnp.sum((pend <= bstart).astype(jnp.int32), axis=-1, keepdims=True)
    eid_col = jnp.minimum(n_before, E - 1)
    eid_ref[...] = eid_col
    total = jnp.sum(pad_cnt, axis=-1, keepdims=True).astype(jnp.int32)
    bstart1 = lax.broadcasted_iota(jnp.int32, (NB, 1), 0) * BLK
    lb_ref[...] = (bstart1 < total).astype(jnp.int32)

    i8 = (lax.broadcasted_iota(jnp.int32, (E, E), 0)
          == lax.broadcasted_iota(jnp.int32, (E, E), 1)).astype(jnp.float32)
    pend_col = lax.dot_general(i8, (off + pad_cnt), (((1,), (1,)), ((), ())),
                               precision=_HI,
                               preferred_element_type=jnp.float32).astype(jnp.int32)
    bstart_r = lax.broadcasted_iota(jnp.int32, (E, NB), 1) * BLK
    nb_row = jnp.sum((pend_col <= bstart_r).astype(jnp.int32), axis=0, keepdims=True)
    eid_row = jnp.minimum(nb_row, E - 1)
    bigger = jnp.where(eid_col > eid_row, eid_col, E)
    nxt_ref[...] = jnp.min(bigger, axis=0, keepdims=True)


def _ffn_kernel(eid_ref, lb_ref, nxt_ref, x_ref, w1_hbm, w2_hbm, y_ref,
                w1a, w2a, w1b, w2b, slot_ref, s1a, s2a, s1b, s2b):
    i = pl.program_id(0)
    e_cur = eid_ref[i, 0]
    e_prv = eid_ref[jnp.maximum(i - 1, 0), 0]
    prev_s = slot_ref[0]
    s = jnp.where(i == 0, 0, jnp.where(e_cur != e_prv, 1 - prev_s, prev_s))
    slot_ref[0] = s
    chg = (i == 0) | (e_cur != e_prv)
    e_nxt = nxt_ref[0, i]
    pref = chg & (e_nxt < E)

    @pl.when(i == 0)
    def _():
        pltpu.make_async_copy(w1_hbm.at[e_cur], w1a, s1a).start()
        pltpu.make_async_copy(w2_hbm.at[e_cur], w2a, s2a).start()

    @pl.when(chg & (s == 0))
    def _():
        pltpu.make_async_copy(w1_hbm.at[e_cur], w1a, s1a).wait()
        pltpu.make_async_copy(w2_hbm.at[e_cur], w2a, s2a).wait()

    @pl.when(chg & (s == 1))
    def _():
        pltpu.make_async_copy(w1_hbm.at[e_cur], w1b, s1b).wait()
        pltpu.make_async_copy(w2_hbm.at[e_cur], w2b, s2b).wait()

    @pl.when(pref & (s == 0))
    def _():
        pltpu.make_async_copy(w1_hbm.at[e_nxt], w1b, s1b).start()
        pltpu.make_async_copy(w2_hbm.at[e_nxt], w2b, s2b).start()

    @pl.when(pref & (s == 1))
    def _():
        pltpu.make_async_copy(w1_hbm.at[e_nxt], w1a, s1a).start()
        pltpu.make_async_copy(w2_hbm.at[e_nxt], w2a, s2a).start()

    live = lb_ref[i, 0] == 1

    @pl.when(live & (s == 0))
    def _():
        h = lax.dot_general(x_ref[...], w1a[...], (((1,), (0,)), ((), ())),
                            preferred_element_type=jnp.float32)
        h = jnp.maximum(h, 0.0)
        y_ref[...] = lax.dot_general(h, w2a[...], (((1,), (0,)), ((), ())),
                                     preferred_element_type=jnp.float32)

    @pl.when(live & (s == 1))
    def _():
        h = lax.dot_general(x_ref[...], w1b[...], (((1,), (0,)), ((), ())),
                            preferred_element_type=jnp.float32)
        h = jnp.maximum(h, 0.0)
        y_ref[...] = lax.dot_general(h, w2b[...], (((1,), (0,)), ((), ())),
                                     preferred_element_type=jnp.float32)


def _combine_kernel(y0_ref, y1_ref, g0_ref, g1_ref, out_ref):
    out_ref[...] = g0_ref[...] * y0_ref[...] + g1_ref[...] * y1_ref[...]


def _sc_mesh():
    return plsc.VectorSubcoreMesh(core_axis_name="c", subcore_axis_name="s",
                                  num_cores=NC, num_subcores=NS)


def _dispatch_body(flat_hbm, d0_hbm, d1_hbm, xs_hbm, x_v, i0_v, i1_v, s0, s1):
    wid = lax.axis_index("s") * NC + lax.axis_index("c")
    base = wid * TPW
    pltpu.sync_copy(flat_hbm.at[pl.ds(base, TPW)], x_v)
    pltpu.sync_copy(d0_hbm.at[pl.ds(base, TPW)], i0_v)
    pltpu.sync_copy(d1_hbm.at[pl.ds(base, TPW)], i1_v)
    cp0 = pltpu.async_copy(x_v, xs_hbm.at[i0_v], s0)
    cp1 = pltpu.async_copy(x_v, xs_hbm.at[i1_v], s1)
    cp0.wait()
    cp1.wait()


def _sc_dispatch(flat, d0, d1):
    k = pl.kernel(
        _dispatch_body,
        out_type=jax.ShapeDtypeStruct((P, D_MODEL), jnp.float32),
        mesh=_sc_mesh(),
        scratch_types=[
            pltpu.VMEM((TPW, D_MODEL), jnp.float32),
            pltpu.VMEM((TPW,), jnp.int32),
            pltpu.VMEM((TPW,), jnp.int32),
            pltpu.SemaphoreType.DMA,
            pltpu.SemaphoreType.DMA,
        ],
    )
    return k(flat, d0, d1)


def _combine_body(ys_hbm, d0_hbm, d1_hbm, y0_hbm, y1_hbm, rows_v, idx_v, sem):
    wid = lax.axis_index("s") * NC + lax.axis_index("c")
    base = wid * TPW
    for c in range(TPW // CH):
        off = base + c * CH
        pltpu.sync_copy(d0_hbm.at[pl.ds(off, CH)], idx_v)
        pltpu.async_copy(ys_hbm.at[idx_v], rows_v, sem).wait()
        pltpu.sync_copy(rows_v, y0_hbm.at[pl.ds(off, CH)])
        pltpu.sync_copy(d1_hbm.at[pl.ds(off, CH)], idx_v)
        pltpu.async_copy(ys_hbm.at[idx_v], rows_v, sem).wait()
        pltpu.sync_copy(rows_v, y1_hbm.at[pl.ds(off, CH)])


def _sc_combine(ys, d0, d1):
    k = pl.kernel(
        _combine_body,
        out_type=(jax.ShapeDtypeStruct((T, D_MODEL), jnp.float32),
                  jax.ShapeDtypeStruct((T, D_MODEL), jnp.float32)),
        mesh=_sc_mesh(),
        scratch_types=[
            pltpu.VMEM((CH, D_MODEL), jnp.float32),
            pltpu.VMEM((CH,), jnp.int32),
            pltpu.SemaphoreType.DMA,
        ],
    )
    return k(ys, d0, d1)


def kernel(hidden_states, router_w, w1, w2):
    b, s, d = hidden_states.shape
    flat = hidden_states.reshape(T, d)

    d0, d1, g0, g1, eid, lb, nxt = pl.pallas_call(
        _route_kernel,
        in_specs=[
            pl.BlockSpec((T, d), lambda: (0, 0)),
            pl.BlockSpec((d, E), lambda: (0, 0)),
        ],
        out_specs=[
            pl.BlockSpec((T, 1), lambda: (0, 0)),
            pl.BlockSpec((T, 1), lambda: (0, 0)),
            pl.BlockSpec((T, 1), lambda: (0, 0)),
            pl.BlockSpec((T, 1), lambda: (0, 0)),
            pl.BlockSpec((NB, 1), lambda: (0, 0)),
            pl.BlockSpec((NB, 1), lambda: (0, 0)),
            pl.BlockSpec((1, NB), lambda: (0, 0)),
        ],
        out_shape=[
            jax.ShapeDtypeStruct((T, 1), jnp.int32),
            jax.ShapeDtypeStruct((T, 1), jnp.int32),
            jax.ShapeDtypeStruct((T, 1), jnp.float32),
            jax.ShapeDtypeStruct((T, 1), jnp.float32),
            jax.ShapeDtypeStruct((NB, 1), jnp.int32),
            jax.ShapeDtypeStruct((NB, 1), jnp.int32),
            jax.ShapeDtypeStruct((1, NB), jnp.int32),
        ],
    )(flat, router_w)

    d0f = d0.reshape(T)
    d1f = d1.reshape(T)

    xs = _sc_dispatch(flat, d0f, d1f)

    grid_spec = pltpu.PrefetchScalarGridSpec(
        num_scalar_prefetch=3,
        grid=(NB,),
        in_specs=[
            pl.BlockSpec((BLK, d), lambda i, eid_r, lb_r, nxt_r: (i, 0)),
            pl.BlockSpec(memory_space=pl.ANY),
            pl.BlockSpec(memory_space=pl.ANY),
        ],
        out_specs=pl.BlockSpec((BLK, d), lambda i, eid_r, lb_r, nxt_r: (i, 0)),
        scratch_shapes=[
            pltpu.VMEM((D_MODEL, D_FF), jnp.float32),
            pltpu.VMEM((D_FF, D_MODEL), jnp.float32),
            pltpu.VMEM((D_MODEL, D_FF), jnp.float32),
            pltpu.VMEM((D_FF, D_MODEL), jnp.float32),
            pltpu.SMEM((1,), jnp.int32),
            pltpu.SemaphoreType.DMA,
            pltpu.SemaphoreType.DMA,
            pltpu.SemaphoreType.DMA,
            pltpu.SemaphoreType.DMA,
        ],
    )
    ys = pl.pallas_call(
        _ffn_kernel,
        grid_spec=grid_spec,
        out_shape=jax.ShapeDtypeStruct((P, d), jnp.float32),
        compiler_params=pltpu.CompilerParams(
            dimension_semantics=("arbitrary",),
        ),
    )(eid, lb, nxt, xs, w1, w2)

    y0, y1 = _sc_combine(ys, d0f, d1f)

    BT = 512
    out = pl.pallas_call(
        _combine_kernel,
        grid=(T // BT,),
        in_specs=[
            pl.BlockSpec((BT, d), lambda t: (t, 0)),
            pl.BlockSpec((BT, d), lambda t: (t, 0)),
            pl.BlockSpec((BT, 1), lambda t: (t, 0)),
            pl.BlockSpec((BT, 1), lambda t: (t, 0)),
        ],
        out_specs=pl.BlockSpec((BT, d), lambda t: (t, 0)),
        out_shape=jax.ShapeDtypeStruct((T, d), jnp.float32),
    )(y0, y1, g0, g1)

    return out.reshape(b, s, d)


# speedup vs baseline: 1.4682x; 1.0330x over previous
import functools

import jax
import jax.numpy as jnp
from jax import lax
from jax.experimental import pallas as pl
from jax.experimental.pallas import tpu as pltpu
from jax.experimental.pallas import tpu_sc as plsc

E = 8
TOP_K = 2
D_MODEL = 1024
D_FF = 2048
T = 2048
BLK = 256
NB = 24
P = NB * BLK
NEG_INF = -1e30

NC = 2
NS = 16
NW = NC * NS
TPW = T // NW
CH = 32

_HI = jax.lax.Precision.HIGHEST


def _route_kernel(x_ref, rw_ref, d0_ref, d1_ref, g0_ref, g1_ref, eid_ref, lb_ref,
                  nxt_ref):
    x = x_ref[...]
    logits = lax.dot_general(x, rw_ref[...], (((1,), (0,)), ((), ())),
                             preferred_element_type=jnp.float32)
    eids = lax.broadcasted_iota(jnp.int32, (T, E), 1)
    m1 = jnp.max(logits, axis=-1, keepdims=True)
    e1 = jnp.min(jnp.where(logits >= m1, eids, E), axis=-1, keepdims=True)
    l2 = jnp.where(eids == e1, NEG_INF, logits)
    m2 = jnp.max(l2, axis=-1, keepdims=True)
    e2 = jnp.min(jnp.where(l2 >= m2, eids, E), axis=-1, keepdims=True)
    r = jnp.exp(m2 - m1)
    g0_ref[...] = 1.0 / (1.0 + r)
    g1_ref[...] = r / (1.0 + r)

    oh0 = (eids == e1).astype(jnp.float32)
    oh1 = (eids == e2).astype(jnp.float32)
    oh = jnp.concatenate([oh0, oh1], axis=1)
    tri = (lax.broadcasted_iota(jnp.int32, (256, 256), 0)
           >= lax.broadcasted_iota(jnp.int32, (256, 256), 1)).astype(jnp.float32)
    parts = []
    carry = jnp.zeros((1, 2 * E), jnp.float32)
    for k in range(T // 256):
        p = lax.dot_general(tri, oh[k * 256:(k + 1) * 256], (((1,), (0,)), ((), ())),
                            precision=_HI, preferred_element_type=jnp.float32) + carry
        parts.append(p)
        carry = p[-1:, :]
    c = jnp.concatenate(parts, axis=0)
    c0 = c[:, :E]
    c1 = c[:, E:]
    c0ex = c0 - oh0
    c1ex = c1 - oh1

    cnt = carry[:, :E] + carry[:, E:]
    pad_cnt = (((cnt.astype(jnp.int32) + (BLK - 1)) >> 8) << 8).astype(jnp.float32)
    m8 = (lax.broadcasted_iota(jnp.int32, (E, E), 0)
          < lax.broadcasted_iota(jnp.int32, (E, E), 1)).astype(jnp.float32)
    off = lax.dot_general(pad_cnt, m8, (((1,), (0,)), ((), ())),
                          precision=_HI, preferred_element_type=jnp.float32)

    rank0 = c0ex + c1ex
    rank1 = c0 + c1ex
    d0c = jnp.sum(oh0 * (off + rank0), axis=-1, keepdims=True)
    d1c = jnp.sum(oh1 * (off + rank1), axis=-1, keepdims=True)
    i128 = (lax.broadcasted_iota(jnp.int32, (128, 128), 0)
            == lax.broadcasted_iota(jnp.int32, (128, 128), 1)).astype(jnp.float32)
    r0, r1 = [], []
    for k in range(T // 128):
        r0.append(lax.dot_general(d0c[k * 128:(k + 1) * 128], i128,
                                  (((0,), (0,)), ((), ())), precision=_HI,
                                  preferred_element_type=jnp.float32))
        r1.append(lax.dot_general(d1c[k * 128:(k + 1) * 128], i128,
                                  (((0,), (0,)), ((), ())), precision=_HI,
                                  preferred_element_type=jnp.float32))
    d0_ref[...] = jnp.concatenate(r0, axis=0).astype(jnp.int32)
    d1_ref[...] = jnp.concatenate(r1, axis=0).astype(jnp.int32)

    pend = (off + pad_cnt).astype(jnp.int32)
    bstart = lax.broadcasted_iota(jnp.int32, (NB, E), 0) * BLK
    n_before = jnp.sum((pend <= bstart).astype(jnp.int32), axis=-1, keepdims=True)
    eid_col = jnp.minimum(n_before, E - 1)
    eid_ref[...] = eid_col
    total = jnp.sum(pad_cnt, axis=-1, keepdims=True).astype(jnp.int32)
    bstart1 = lax.broadcasted_iota(jnp.int32, (NB, 1), 0) * BLK
    lb_ref[...] = (bstart1 < total).astype(jnp.int32)

    i8 = (lax.broadcasted_iota(jnp.int32, (E, E), 0)
          == lax.broadcasted_iota(jnp.int32, (E, E), 1)).astype(jnp.float32)
    pend_col = lax.dot_general(i8, (off + pad_cnt), (((1,), (1,)), ((), ())),
                               precision=_HI,
                               preferred_element_type=jnp.float32).astype(jnp.int32)
    bstart_r = lax.broadcasted_iota(jnp.int32, (E, NB), 1) * BLK
    nb_row = jnp.sum((pend_col <= bstart_r).astype(jnp.int32), axis=0, keepdims=True)
    eid_row = jnp.minimum(nb_row, E - 1)
    bigger = jnp.where(eid_col > eid_row, eid_col, E)
    nxt_ref[...] = jnp.min(bigger, axis=0, keepdims=True)


def _ffn_kernel(eid_ref, lb_ref, nxt_ref, x_ref, w1_hbm, w2_hbm, y_ref,
                w1a, w2a, w1b, w2b, slot_ref, s1a, s2a, s1b, s2b):
    i = pl.program_id(0)
    e_cur = eid_ref[i, 0]
    e_prv = eid_ref[jnp.maximum(i - 1, 0), 0]
    prev_s = slot_ref[0]
    s = jnp.where(i == 0, 0, jnp.where(e_cur != e_prv, 1 - prev_s, prev_s))
    slot_ref[0] = s
    chg = (i == 0) | (e_cur != e_prv)
    e_nxt = nxt_ref[0, i]
    pref = chg & (e_nxt < E)

    @pl.when(i == 0)
    def _():
        pltpu.make_async_copy(w1_hbm.at[e_cur], w1a, s1a).start()
        pltpu.make_async_copy(w2_hbm.at[e_cur], w2a, s2a).start()

    @pl.when(chg & (s == 0))
    def _():
        pltpu.make_async_copy(w1_hbm.at[e_cur], w1a, s1a).wait()
        pltpu.make_async_copy(w2_hbm.at[e_cur], w2a, s2a).wait()

    @pl.when(chg & (s == 1))
    def _():
        pltpu.make_async_copy(w1_hbm.at[e_cur], w1b, s1b).wait()
        pltpu.make_async_copy(w2_hbm.at[e_cur], w2b, s2b).wait()

    @pl.when(pref & (s == 0))
    def _():
        pltpu.make_async_copy(w1_hbm.at[e_nxt], w1b, s1b).start()
        pltpu.make_async_copy(w2_hbm.at[e_nxt], w2b, s2b).start()

    @pl.when(pref & (s == 1))
    def _():
        pltpu.make_async_copy(w1_hbm.at[e_nxt], w1a, s1a).start()
        pltpu.make_async_copy(w2_hbm.at[e_nxt], w2a, s2a).start()

    live = lb_ref[i, 0] == 1

    @pl.when(live & (s == 0))
    def _():
        h = lax.dot_general(x_ref[...], w1a[...], (((1,), (0,)), ((), ())),
                            preferred_element_type=jnp.float32)
        h = jnp.maximum(h, 0.0)
        y_ref[...] = lax.dot_general(h, w2a[...], (((1,), (0,)), ((), ())),
                                     preferred_element_type=jnp.float32)

    @pl.when(live & (s == 1))
    def _():
        h = lax.dot_general(x_ref[...], w1b[...], (((1,), (0,)), ((), ())),
                            preferred_element_type=jnp.float32)
        h = jnp.maximum(h, 0.0)
        y_ref[...] = lax.dot_general(h, w2b[...], (((1,), (0,)), ((), ())),
                                     preferred_element_type=jnp.float32)


def _combine_kernel(y0_ref, y1_ref, g0_ref, g1_ref, out_ref):
    out_ref[...] = g0_ref[...] * y0_ref[...] + g1_ref[...] * y1_ref[...]


def _sc_mesh():
    return plsc.VectorSubcoreMesh(core_axis_name="c", subcore_axis_name="s",
                                  num_cores=NC, num_subcores=NS)


def _dispatch_body(flat_hbm, d0_hbm, d1_hbm, xs_hbm, x_v, i0_v, i1_v, s0, s1):
    wid = lax.axis_index("s") * NC + lax.axis_index("c")
    base = wid * TPW
    pltpu.sync_copy(flat_hbm.at[pl.ds(base, TPW)], x_v)
    pltpu.sync_copy(d0_hbm.at[pl.ds(base, TPW)], i0_v)
    pltpu.sync_copy(d1_hbm.at[pl.ds(base, TPW)], i1_v)
    cp0 = pltpu.async_copy(x_v, xs_hbm.at[i0_v], s0)
    cp1 = pltpu.async_copy(x_v, xs_hbm.at[i1_v], s1)
    cp0.wait()
    cp1.wait()


def _sc_dispatch(flat, d0, d1):
    k = pl.kernel(
        _dispatch_body,
        out_type=jax.ShapeDtypeStruct((P, D_MODEL), jnp.float32),
        mesh=_sc_mesh(),
        scratch_types=[
            pltpu.VMEM((TPW, D_MODEL), jnp.float32),
            pltpu.VMEM((TPW,), jnp.int32),
            pltpu.VMEM((TPW,), jnp.int32),
            pltpu.SemaphoreType.DMA,
            pltpu.SemaphoreType.DMA,
        ],
    )
    return k(flat, d0, d1)


def _combine_body(ys_hbm, d0_hbm, d1_hbm, y0_hbm, y1_hbm,
                  r0_v, r1_v, i0_v, i1_v, g0s, g1s, w0s, w1s):
    wid = lax.axis_index("s") * NC + lax.axis_index("c")
    base = wid * TPW
    for c in range(TPW // CH):
        off = base + c * CH
        pltpu.sync_copy(d0_hbm.at[pl.ds(off, CH)], i0_v)
        pltpu.sync_copy(d1_hbm.at[pl.ds(off, CH)], i1_v)
        cp0 = pltpu.async_copy(ys_hbm.at[i0_v], r0_v, g0s)
        cp1 = pltpu.async_copy(ys_hbm.at[i1_v], r1_v, g1s)
        cp0.wait()
        wr0 = pltpu.async_copy(r0_v, y0_hbm.at[pl.ds(off, CH)], w0s)
        cp1.wait()
        wr1 = pltpu.async_copy(r1_v, y1_hbm.at[pl.ds(off, CH)], w1s)
        wr0.wait()
        wr1.wait()


def _sc_combine(ys, d0, d1):
    k = pl.kernel(
        _combine_body,
        out_type=(jax.ShapeDtypeStruct((T, D_MODEL), jnp.float32),
                  jax.ShapeDtypeStruct((T, D_MODEL), jnp.float32)),
        mesh=_sc_mesh(),
        scratch_types=[
            pltpu.VMEM((CH, D_MODEL), jnp.float32),
            pltpu.VMEM((CH, D_MODEL), jnp.float32),
            pltpu.VMEM((CH,), jnp.int32),
            pltpu.VMEM((CH,), jnp.int32),
            pltpu.SemaphoreType.DMA,
            pltpu.SemaphoreType.DMA,
            pltpu.SemaphoreType.DMA,
            pltpu.SemaphoreType.DMA,
        ],
    )
    return k(ys, d0, d1)


def kernel(hidden_states, router_w, w1, w2):
    b, s, d = hidden_states.shape
    flat = hidden_states.reshape(T, d)

    d0, d1, g0, g1, eid, lb, nxt = pl.pallas_call(
        _route_kernel,
        in_specs=[
            pl.BlockSpec((T, d), lambda: (0, 0)),
            pl.BlockSpec((d, E), lambda: (0, 0)),
        ],
        out_specs=[
            pl.BlockSpec((T // 128, 128), lambda: (0, 0)),
            pl.BlockSpec((T // 128, 128), lambda: (0, 0)),
            pl.BlockSpec((T, 1), lambda: (0, 0)),
            pl.BlockSpec((T, 1), lambda: (0, 0)),
            pl.BlockSpec((NB, 1), lambda: (0, 0)),
            pl.BlockSpec((NB, 1), lambda: (0, 0)),
            pl.BlockSpec((1, NB), lambda: (0, 0)),
        ],
        out_shape=[
            jax.ShapeDtypeStruct((T // 128, 128), jnp.int32),
            jax.ShapeDtypeStruct((T // 128, 128), jnp.int32),
            jax.ShapeDtypeStruct((T, 1), jnp.float32),
            jax.ShapeDtypeStruct((T, 1), jnp.float32),
            jax.ShapeDtypeStruct((NB, 1), jnp.int32),
            jax.ShapeDtypeStruct((NB, 1), jnp.int32),
            jax.ShapeDtypeStruct((1, NB), jnp.int32),
        ],
    )(flat, router_w)

    d0f = d0.reshape(T)
    d1f = d1.reshape(T)

    xs = _sc_dispatch(flat, d0f, d1f)

    grid_spec = pltpu.PrefetchScalarGridSpec(
        num_scalar_prefetch=3,
        grid=(NB,),
        in_specs=[
            pl.BlockSpec((BLK, d), lambda i, eid_r, lb_r, nxt_r: (i, 0)),
            pl.BlockSpec(memory_space=pl.ANY),
            pl.BlockSpec(memory_space=pl.ANY),
        ],
        out_specs=pl.BlockSpec((BLK, d), lambda i, eid_r, lb_r, nxt_r: (i, 0)),
        scratch_shapes=[
            pltpu.VMEM((D_MODEL, D_FF), jnp.float32),
            pltpu.VMEM((D_FF, D_MODEL), jnp.float32),
            pltpu.VMEM((D_MODEL, D_FF), jnp.float32),
            pltpu.VMEM((D_FF, D_MODEL), jnp.float32),
            pltpu.SMEM((1,), jnp.int32),
            pltpu.SemaphoreType.DMA,
            pltpu.SemaphoreType.DMA,
            pltpu.SemaphoreType.DMA,
            pltpu.SemaphoreType.DMA,
        ],
    )
    ys = pl.pallas_call(
        _ffn_kernel,
        grid_spec=grid_spec,
        out_shape=jax.ShapeDtypeStruct((P, d), jnp.float32),
        compiler_params=pltpu.CompilerParams(
            dimension_semantics=("arbitrary",),
        ),
    )(eid, lb, nxt, xs, w1, w2)

    y0, y1 = _sc_combine(ys, d0f, d1f)

    BT = 512
    out = pl.pallas_call(
        _combine_kernel,
        grid=(T // BT,),
        in_specs=[
            pl.BlockSpec((BT, d), lambda t: (t, 0)),
            pl.BlockSpec((BT, d), lambda t: (t, 0)),
            pl.BlockSpec((BT, 1), lambda t: (t, 0)),
            pl.BlockSpec((BT, 1), lambda t: (t, 0)),
        ],
        out_specs=pl.BlockSpec((BT, d), lambda t: (t, 0)),
        out_shape=jax.ShapeDtypeStruct((T, d), jnp.float32),
    )(y0, y1, g0, g1)

    return out.reshape(b, s, d)


# speedup vs baseline: 1.4752x; 1.0048x over previous
import functools

import jax
import jax.numpy as jnp
from jax import lax
from jax.experimental import pallas as pl
from jax.experimental.pallas import tpu as pltpu
from jax.experimental.pallas import tpu_sc as plsc

E = 8
TOP_K = 2
D_MODEL = 1024
D_FF = 2048
T = 2048
BLK = 256
NB = 24
P = NB * BLK
NEG_INF = -1e30

NC = 2
NS = 16
NW = NC * NS
TPW = T // NW
CH = 32

_HI = jax.lax.Precision.HIGHEST


def _route_kernel(x_ref, rw_ref, d0_ref, d1_ref, g0_ref, g1_ref, eid_ref, lb_ref,
                  nxt_ref):
    x = x_ref[...]
    logits = lax.dot_general(x, rw_ref[...], (((1,), (0,)), ((), ())),
                             preferred_element_type=jnp.float32)
    eids = lax.broadcasted_iota(jnp.int32, (T, E), 1)
    m1 = jnp.max(logits, axis=-1, keepdims=True)
    e1 = jnp.min(jnp.where(logits >= m1, eids, E), axis=-1, keepdims=True)
    l2 = jnp.where(eids == e1, NEG_INF, logits)
    m2 = jnp.max(l2, axis=-1, keepdims=True)
    e2 = jnp.min(jnp.where(l2 >= m2, eids, E), axis=-1, keepdims=True)
    r = jnp.exp(m2 - m1)
    g0_ref[...] = 1.0 / (1.0 + r)
    g1_ref[...] = r / (1.0 + r)

    oh0 = (eids == e1).astype(jnp.float32)
    oh1 = (eids == e2).astype(jnp.float32)
    oh = jnp.concatenate([oh0, oh1], axis=1)
    tri = (lax.broadcasted_iota(jnp.int32, (256, 256), 0)
           >= lax.broadcasted_iota(jnp.int32, (256, 256), 1)).astype(jnp.float32)
    parts = []
    carry = jnp.zeros((1, 2 * E), jnp.float32)
    for k in range(T // 256):
        p = lax.dot_general(tri, oh[k * 256:(k + 1) * 256], (((1,), (0,)), ((), ())),
                            precision=_HI, preferred_element_type=jnp.float32) + carry
        parts.append(p)
        carry = p[-1:, :]
    c = jnp.concatenate(parts, axis=0)
    c0 = c[:, :E]
    c1 = c[:, E:]
    c0ex = c0 - oh0
    c1ex = c1 - oh1

    cnt = carry[:, :E] + carry[:, E:]
    pad_cnt = (((cnt.astype(jnp.int32) + (BLK - 1)) >> 8) << 8).astype(jnp.float32)
    m8 = (lax.broadcasted_iota(jnp.int32, (E, E), 0)
          < lax.broadcasted_iota(jnp.int32, (E, E), 1)).astype(jnp.float32)
    off = lax.dot_general(pad_cnt, m8, (((1,), (0,)), ((), ())),
                          precision=_HI, preferred_element_type=jnp.float32)

    rank0 = c0ex + c1ex
    rank1 = c0 + c1ex
    d0c = jnp.sum(oh0 * (off + rank0), axis=-1, keepdims=True)
    d1c = jnp.sum(oh1 * (off + rank1), axis=-1, keepdims=True)
    i128 = (lax.broadcasted_iota(jnp.int32, (128, 128), 0)
            == lax.broadcasted_iota(jnp.int32, (128, 128), 1)).astype(jnp.float32)
    r0, r1 = [], []
    for k in range(T // 128):
        r0.append(lax.dot_general(d0c[k * 128:(k + 1) * 128], i128,
                                  (((0,), (0,)), ((), ())), precision=_HI,
                                  preferred_element_type=jnp.float32))
        r1.append(lax.dot_general(d1c[k * 128:(k + 1) * 128], i128,
                                  (((0,), (0,)), ((), ())), precision=_HI,
                                  preferred_element_type=jnp.float32))
    d0_ref[...] = jnp.concatenate(r0, axis=0).astype(jnp.int32)
    d1_ref[...] = jnp.concatenate(r1, axis=0).astype(jnp.int32)

    pend = (off + pad_cnt).astype(jnp.int32)
    bstart = lax.broadcasted_iota(jnp.int32, (NB, E), 0) * BLK
    n_before = jnp.sum((pend <= bstart).astype(jnp.int32), axis=-1, keepdims=True)
    eid_col = jnp.minimum(n_before, E - 1)
    eid_ref[...] = eid_col
    total = jnp.sum(pad_cnt, axis=-1, keepdims=True).astype(jnp.int32)
    bstart1 = lax.broadcasted_iota(jnp.int32, (NB, 1), 0) * BLK
    lb_ref[...] = (bstart1 < total).astype(jnp.int32)

    i8 = (lax.broadcasted_iota(jnp.int32, (E, E), 0)
          == lax.broadcasted_iota(jnp.int32, (E, E), 1)).astype(jnp.float32)
    pend_col = lax.dot_general(i8, (off + pad_cnt), (((1,), (1,)), ((), ())),
                               precision=_HI,
                               preferred_element_type=jnp.float32).astype(jnp.int32)
    bstart_r = lax.broadcasted_iota(jnp.int32, (E, NB), 1) * BLK
    nb_row = jnp.sum((pend_col <= bstart_r).astype(jnp.int32), axis=0, keepdims=True)
    eid_row = jnp.minimum(nb_row, E - 1)
    bigger = jnp.where(eid_col > eid_row, eid_col, E)
    nxt_ref[...] = jnp.min(bigger, axis=0, keepdims=True)


def _ffn_kernel(eid_ref, lb_ref, nxt_ref, x_ref, w1_hbm, w2_hbm, y_ref,
                w1a, w2a, w1b, w2b, slot_ref, s1a, s2a, s1b, s2b):
    i = pl.program_id(0)
    e_cur = eid_ref[i, 0]
    e_prv = eid_ref[jnp.maximum(i - 1, 0), 0]
    prev_s = slot_ref[0]
    s = jnp.where(i == 0, 0, jnp.where(e_cur != e_prv, 1 - prev_s, prev_s))
    slot_ref[0] = s
    chg = (i == 0) | (e_cur != e_prv)
    e_nxt = nxt_ref[0, i]
    pref = chg & (e_nxt < E)

    @pl.when(i == 0)
    def _():
        pltpu.make_async_copy(w1_hbm.at[e_cur], w1a, s1a).start()
        pltpu.make_async_copy(w2_hbm.at[e_cur], w2a, s2a).start()

    @pl.when(pref & (s == 0))
    def _():
        pltpu.make_async_copy(w1_hbm.at[e_nxt], w1b, s1b).start()
        pltpu.make_async_copy(w2_hbm.at[e_nxt], w2b, s2b).start()

    @pl.when(pref & (s == 1))
    def _():
        pltpu.make_async_copy(w1_hbm.at[e_nxt], w1a, s1a).start()
        pltpu.make_async_copy(w2_hbm.at[e_nxt], w2a, s2a).start()

    live = lb_ref[i, 0] == 1

    @pl.when(live & (s == 0))
    def _():
        @pl.when(chg)
        def _():
            pltpu.make_async_copy(w1_hbm.at[e_cur], w1a, s1a).wait()

        h = lax.dot_general(x_ref[...], w1a[...], (((1,), (0,)), ((), ())),
                            preferred_element_type=jnp.float32)
        h = jnp.maximum(h, 0.0)

        @pl.when(chg)
        def _():
            pltpu.make_async_copy(w2_hbm.at[e_cur], w2a, s2a).wait()

        y_ref[...] = lax.dot_general(h, w2a[...], (((1,), (0,)), ((), ())),
                                     preferred_element_type=jnp.float32)

    @pl.when(live & (s == 1))
    def _():
        @pl.when(chg)
        def _():
            pltpu.make_async_copy(w1_hbm.at[e_cur], w1b, s1b).wait()

        h = lax.dot_general(x_ref[...], w1b[...], (((1,), (0,)), ((), ())),
                            preferred_element_type=jnp.float32)
        h = jnp.maximum(h, 0.0)

        @pl.when(chg)
        def _():
            pltpu.make_async_copy(w2_hbm.at[e_cur], w2b, s2b).wait()

        y_ref[...] = lax.dot_general(h, w2b[...], (((1,), (0,)), ((), ())),
                                     preferred_element_type=jnp.float32)

    @pl.when((~live) & chg & (s == 0))
    def _():
        pltpu.make_async_copy(w1_hbm.at[e_cur], w1a, s1a).wait()
        pltpu.make_async_copy(w2_hbm.at[e_cur], w2a, s2a).wait()

    @pl.when((~live) & chg & (s == 1))
    def _():
        pltpu.make_async_copy(w1_hbm.at[e_cur], w1b, s1b).wait()
        pltpu.make_async_copy(w2_hbm.at[e_cur], w2b, s2b).wait()


def _combine_kernel(y0_ref, y1_ref, g0_ref, g1_ref, out_ref):
    out_ref[0] = g0_ref[...] * y0_ref[...] + g1_ref[...] * y1_ref[...]


def _sc_mesh():
    return plsc.VectorSubcoreMesh(core_axis_name="c", subcore_axis_name="s",
                                  num_cores=NC, num_subcores=NS)


def _dispatch_body(flat_hbm, d0_hbm, d1_hbm, xs_hbm, x_v, i0_v, i1_v, s0, s1):
    wid = lax.axis_index("s") * NC + lax.axis_index("c")
    base = wid * TPW
    pltpu.sync_copy(flat_hbm.at[pl.ds(base, TPW)], x_v)
    pltpu.sync_copy(d0_hbm.at[pl.ds(base, TPW)], i0_v)
    pltpu.sync_copy(d1_hbm.at[pl.ds(base, TPW)], i1_v)
    cp0 = pltpu.async_copy(x_v, xs_hbm.at[i0_v], s0)
    cp1 = pltpu.async_copy(x_v, xs_hbm.at[i1_v], s1)
    cp0.wait()
    cp1.wait()


def _sc_dispatch(flat, d0, d1):
    k = pl.kernel(
        _dispatch_body,
        out_type=jax.ShapeDtypeStruct((P, D_MODEL), jnp.float32),
        mesh=_sc_mesh(),
        scratch_types=[
            pltpu.VMEM((TPW, D_MODEL), jnp.float32),
            pltpu.VMEM((TPW,), jnp.int32),
            pltpu.VMEM((TPW,), jnp.int32),
            pltpu.SemaphoreType.DMA,
            pltpu.SemaphoreType.DMA,
        ],
    )
    return k(flat, d0, d1)


def _combine_body(ys_hbm, d0_hbm, d1_hbm, y0_hbm, y1_hbm,
                  r0_v, r1_v, i0_v, i1_v, g0s, g1s, w0s, w1s):
    wid = lax.axis_index("s") * NC + lax.axis_index("c")
    base = wid * TPW
    for c in range(TPW // CH):
        off = base + c * CH
        pltpu.sync_copy(d0_hbm.at[pl.ds(off, CH)], i0_v)
        pltpu.sync_copy(d1_hbm.at[pl.ds(off, CH)], i1_v)
        cp0 = pltpu.async_copy(ys_hbm.at[i0_v], r0_v, g0s)
        cp1 = pltpu.async_copy(ys_hbm.at[i1_v], r1_v, g1s)
        cp0.wait()
        wr0 = pltpu.async_copy(r0_v, y0_hbm.at[pl.ds(off, CH)], w0s)
        cp1.wait()
        wr1 = pltpu.async_copy(r1_v, y1_hbm.at[pl.ds(off, CH)], w1s)
        wr0.wait()
        wr1.wait()


def _sc_combine(ys, d0, d1):
    k = pl.kernel(
        _combine_body,
        out_type=(jax.ShapeDtypeStruct((T, D_MODEL), jnp.float32),
                  jax.ShapeDtypeStruct((T, D_MODEL), jnp.float32)),
        mesh=_sc_mesh(),
        scratch_types=[
            pltpu.VMEM((CH, D_MODEL), jnp.float32),
            pltpu.VMEM((CH, D_MODEL), jnp.float32),
            pltpu.VMEM((CH,), jnp.int32),
            pltpu.VMEM((CH,), jnp.int32),
            pltpu.SemaphoreType.DMA,
            pltpu.SemaphoreType.DMA,
            pltpu.SemaphoreType.DMA,
            pltpu.SemaphoreType.DMA,
        ],
    )
    return k(ys, d0, d1)


def kernel(hidden_states, router_w, w1, w2):
    b, s, d = hidden_states.shape
    flat = hidden_states.reshape(T, d)

    d0, d1, g0, g1, eid, lb, nxt = pl.pallas_call(
        _route_kernel,
        in_specs=[
            pl.BlockSpec((T, d), lambda: (0, 0)),
            pl.BlockSpec((d, E), lambda: (0, 0)),
        ],
        out_specs=[
            pl.BlockSpec((T // 128, 128), lambda: (0, 0)),
            pl.BlockSpec((T // 128, 128), lambda: (0, 0)),
            pl.BlockSpec((T, 1), lambda: (0, 0)),
            pl.BlockSpec((T, 1), lambda: (0, 0)),
            pl.BlockSpec((NB, 1), lambda: (0, 0)),
            pl.BlockSpec((NB, 1), lambda: (0, 0)),
            pl.BlockSpec((1, NB), lambda: (0, 0)),
        ],
        out_shape=[
            jax.ShapeDtypeStruct((T // 128, 128), jnp.int32),
            jax.ShapeDtypeStruct((T // 128, 128), jnp.int32),
            jax.ShapeDtypeStruct((T, 1), jnp.float32),
            jax.ShapeDtypeStruct((T, 1), jnp.float32),
            jax.ShapeDtypeStruct((NB, 1), jnp.int32),
            jax.ShapeDtypeStruct((NB, 1), jnp.int32),
            jax.ShapeDtypeStruct((1, NB), jnp.int32),
        ],
    )(flat, router_w)

    d0f = d0.reshape(T)
    d1f = d1.reshape(T)

    xs = _sc_dispatch(flat, d0f, d1f)

    grid_spec = pltpu.PrefetchScalarGridSpec(
        num_scalar_prefetch=3,
        grid=(NB,),
        in_specs=[
            pl.BlockSpec((BLK, d), lambda i, eid_r, lb_r, nxt_r: (i, 0)),
            pl.BlockSpec(memory_space=pl.ANY),
            pl.BlockSpec(memory_space=pl.ANY),
        ],
        out_specs=pl.BlockSpec((BLK, d), lambda i, eid_r, lb_r, nxt_r: (i, 0)),
        scratch_shapes=[
            pltpu.VMEM((D_MODEL, D_FF), jnp.float32),
            pltpu.VMEM((D_FF, D_MODEL), jnp.float32),
            pltpu.VMEM((D_MODEL, D_FF), jnp.float32),
            pltpu.VMEM((D_FF, D_MODEL), jnp.float32),
            pltpu.SMEM((1,), jnp.int32),
            pltpu.SemaphoreType.DMA,
            pltpu.SemaphoreType.DMA,
            pltpu.SemaphoreType.DMA,
            pltpu.SemaphoreType.DMA,
        ],
    )
    ys = pl.pallas_call(
        _ffn_kernel,
        grid_spec=grid_spec,
        out_shape=jax.ShapeDtypeStruct((P, d), jnp.float32),
        compiler_params=pltpu.CompilerParams(
            dimension_semantics=("arbitrary",),
        ),
    )(eid, lb, nxt, xs, w1, w2)

    y0, y1 = _sc_combine(ys, d0f, d1f)

    BT = 512
    out = pl.pallas_call(
        _combine_kernel,
        grid=(T // BT,),
        in_specs=[
            pl.BlockSpec((BT, d), lambda t: (t, 0)),
            pl.BlockSpec((BT, d), lambda t: (t, 0)),
            pl.BlockSpec((BT, 1), lambda t: (t, 0)),
            pl.BlockSpec((BT, 1), lambda t: (t, 0)),
        ],
        out_specs=pl.BlockSpec((1, BT, d), lambda t: (0, t, 0)),
        out_shape=jax.ShapeDtypeStruct((1, T, d), jnp.float32),
    )(y0, y1, g0, g1)

    return out.reshape(b, s, d)
